# baseline (speedup 1.0000x reference)
import jax
import jax.numpy as jnp
from jax.experimental import pallas as pl
from jax.experimental.pallas import tpu as pltpu

_N = 10000
_M = 512
_D = 128
_STEPS = 5
_DT = 0.2
_BN = 1000
_NB = _N // _BN


def _sde_kernel(inc_ref, x0_ref, w_ref, b_ref, ls_ref, dw_ref, out_ref,
                incb_ref, inct_ref, e2_ref, eacc_ref, q_ref, c_ref,
                csum_ref, inv_e_ref, inv_v_ref):
    s = pl.program_id(0)
    i = pl.program_id(1)
    row = i * _BN

    @pl.when(s == 0)
    def _prologue():
        inc_f = inc_ref[...]
        x0 = x0_ref[...]
        out_ref[pl.ds(row, _BN), :] = x0
        inv_v_ref[pl.ds(row, _BN), :] = (
            1.0 / (jnp.sum(inc_f, axis=1)[:, None] + 1e-6))
        col = jnp.sum(inc_f, axis=0)[:, None]

        @pl.when(i == 0)
        def _():
            inv_e_ref[...] = col

        @pl.when(i > 0)
        def _():
            inv_e_ref[...] = inv_e_ref[...] + col

        @pl.when(i == _NB - 1)
        def _():
            inv_e_ref[...] = 1.0 / (inv_e_ref[...] + 1e-6)

        inc_bf = inc_f.astype(jnp.bfloat16)
        incb_ref[i] = inc_bf
        inct = jnp.transpose(inc_bf)
        inct_ref[i] = inct

        sigma = jnp.exp(ls_ref[...])
        dwb = dw_ref[0]
        csum_ref[pl.ds(row, _BN), :] = jnp.sum(dwb, axis=1) * sigma
        for j in range(_STEPS):
            qj = jnp.dot(inct, dwb[:, j, :].astype(jnp.bfloat16),
                         preferred_element_type=jnp.float32) * sigma

            @pl.when(i == 0)
            def _(qj=qj, j=j):
                q_ref[j] = qj

            @pl.when(i > 0)
            def _(qj=qj, j=j):
                q_ref[j] = q_ref[j] + qj

        part = jnp.dot(inct, x0.astype(jnp.bfloat16),
                       preferred_element_type=jnp.float32)

        @pl.when(i == 0)
        def _():
            eacc_ref[...] = part

        @pl.when(i > 0)
        def _():
            eacc_ref[...] = eacc_ref[...] + part

        @pl.when(i == _NB - 1)
        def _():
            e2 = jnp.dot(eacc_ref[...] * inv_e_ref[...], w_ref[...],
                         preferred_element_type=jnp.float32) + b_ref[...]
            e2_ref[...] = e2.astype(jnp.bfloat16)
            c_ref[...] = q_ref[0]

    @pl.when(s > 0)
    def _update():
        z = out_ref[pl.ds(row, _BN), :]
        agg = jnp.dot(incb_ref[i], e2_ref[...],
                      preferred_element_type=jnp.float32)
        agg = agg * inv_v_ref[pl.ds(row, _BN), :]
        znew = z + jnp.tanh(agg) * _DT

        @pl.when(s < _STEPS)
        def _():
            out_ref[pl.ds(row, _BN), :] = znew

        @pl.when(s == _STEPS)
        def _():
            out_ref[pl.ds(row, _BN), :] = znew + csum_ref[pl.ds(row, _BN), :]

    @pl.when((s > 0) & (s < _STEPS))
    def _accumulate():
        znew = out_ref[pl.ds(row, _BN), :].astype(jnp.bfloat16)
        part = jnp.dot(inct_ref[i], znew, preferred_element_type=jnp.float32)

        @pl.when(i == 0)
        def _():
            eacc_ref[...] = part

        @pl.when(i > 0)
        def _():
            eacc_ref[...] = eacc_ref[...] + part

        @pl.when(i == _NB - 1)
        def _():
            e2 = jnp.dot((eacc_ref[...] + c_ref[...]) * inv_e_ref[...],
                         w_ref[...],
                         preferred_element_type=jnp.float32) + b_ref[...]
            e2_ref[...] = e2.astype(jnp.bfloat16)
            c_ref[...] = c_ref[...] + q_ref[s]


def kernel(node_features, incidence, W, b, log_sigma, dW):
    out = pl.pallas_call(
        _sde_kernel,
        grid=(_STEPS + 1, _NB),
        in_specs=[
            pl.BlockSpec((_BN, _M),
                         lambda s, i: (jnp.where(s == 0, i, _NB - 1), 0)),
            pl.BlockSpec((_BN, _D),
                         lambda s, i: (jnp.where(s == 0, i, _NB - 1), 0)),
            pl.BlockSpec((_D, _D), lambda s, i: (0, 0)),
            pl.BlockSpec((1, _D), lambda s, i: (0, 0)),
            pl.BlockSpec((1, _D), lambda s, i: (0, 0)),
            pl.BlockSpec((1, _BN, _STEPS, _D),
                         lambda s, i: (jnp.where(s == 0, i, _NB - 1), 0, 0, 0)),
        ],
        out_specs=pl.BlockSpec((_N, _D), lambda s, i: (0, 0)),
        out_shape=jax.ShapeDtypeStruct((_N, _D), jnp.float32),
        scratch_shapes=[
            pltpu.VMEM((_NB, _BN, _M), jnp.bfloat16),
            pltpu.VMEM((_NB, _M, _BN), jnp.bfloat16),
            pltpu.VMEM((_M, _D), jnp.bfloat16),
            pltpu.VMEM((_M, _D), jnp.float32),
            pltpu.VMEM((_STEPS, _M, _D), jnp.float32),
            pltpu.VMEM((_M, _D), jnp.float32),
            pltpu.VMEM((_N, _D), jnp.float32),
            pltpu.VMEM((_M, 1), jnp.float32),
            pltpu.VMEM((_N, 1), jnp.float32),
        ],
        compiler_params=pltpu.CompilerParams(
            dimension_semantics=("arbitrary", "arbitrary"),
            vmem_limit_bytes=100 * 1024 * 1024,
        ),
    )(incidence, node_features, W, b.reshape(1, _D),
      log_sigma.reshape(1, _D),
      dW.reshape(_STEPS, _N, _D).transpose(1, 0, 2).reshape(
          _NB, _BN, _STEPS, _D))
    return out.reshape(1, _N * _D)


# speedup vs baseline: 1.2768x; 1.2768x over previous
import jax
import jax.numpy as jnp
from jax.experimental import pallas as pl
from jax.experimental.pallas import tpu as pltpu

_N = 10000
_M = 512
_D = 128
_STEPS = 5
_DT = 0.2
_BN = 2000
_NB = _N // _BN


def _dw_copy(dw_hbm, stage_ref, sem, blk, buf):
    return [
        pltpu.make_async_copy(
            dw_hbm.at[blk, :, j, :], stage_ref.at[buf, j], sem.at[buf])
        for j in range(_STEPS)
    ]


def _sde_kernel(w_ref, b_ref, ls_ref, inc_ref, x0_ref, dw_hbm, out_ref,
                incb_ref, inct_ref, e2_ref, eacc_ref, q_ref, c_ref,
                csum_ref, inv_e_ref, stage_ref, sem):
    s = pl.program_id(0)
    i = pl.program_id(1)
    row = i * _BN

    @pl.when(s == 0)
    def _prologue():
        @pl.when(i == 0)
        def _():
            for c in _dw_copy(dw_hbm, stage_ref, sem, 0, 0):
                c.start()
            for c in _dw_copy(dw_hbm, stage_ref, sem, 1, 1):
                c.start()

        inc_f = inc_ref[...]
        x0 = x0_ref[...]
        out_ref[pl.ds(row, _BN), :] = x0
        inv_v = 1.0 / (jnp.sum(inc_f, axis=1)[:, None] + 1e-6)
        col = jnp.sum(inc_f, axis=0)[:, None]

        @pl.when(i == 0)
        def _():
            inv_e_ref[...] = col

        @pl.when(i > 0)
        def _():
            inv_e_ref[...] = inv_e_ref[...] + col

        @pl.when(i == _NB - 1)
        def _():
            inv_e_ref[...] = 1.0 / (inv_e_ref[...] + 1e-6)

        incb_ref[i] = (inc_f * inv_v).astype(jnp.bfloat16)
        inct = jnp.transpose(inc_f.astype(jnp.bfloat16))
        inct_ref[i] = inct

        sigma = jnp.exp(ls_ref[...])
        for c in _dw_copy(dw_hbm, stage_ref, sem, i, i % 2):
            c.wait()
        stage = stage_ref[i % 2]
        csum_ref[pl.ds(row, _BN), :] = (
            (((stage[0] + stage[1]) + (stage[2] + stage[3])) + stage[4])
            * sigma)
        for j in range(_STEPS):
            qj = jnp.dot(inct, stage_ref[i % 2, j].astype(jnp.bfloat16),
                         preferred_element_type=jnp.float32) * sigma

            @pl.when(i == 0)
            def _(qj=qj, j=j):
                q_ref[j] = qj

            @pl.when(i > 0)
            def _(qj=qj, j=j):
                q_ref[j] = q_ref[j] + qj

        @pl.when(i + 2 < _NB)
        def _():
            for c in _dw_copy(dw_hbm, stage_ref, sem, i + 2, i % 2):
                c.start()

        part = jnp.dot(inct, x0.astype(jnp.bfloat16),
                       preferred_element_type=jnp.float32)

        @pl.when(i == 0)
        def _():
            eacc_ref[...] = part

        @pl.when(i > 0)
        def _():
            eacc_ref[...] = eacc_ref[...] + part

        @pl.when(i == _NB - 1)
        def _():
            e2 = jnp.dot(eacc_ref[...] * inv_e_ref[...], w_ref[...],
                         preferred_element_type=jnp.float32) + b_ref[...]
            e2_ref[...] = e2.astype(jnp.bfloat16)
            c_ref[...] = q_ref[0]

    @pl.when(s > 0)
    def _update():
        z = out_ref[pl.ds(row, _BN), :]
        agg = jnp.dot(incb_ref[i], e2_ref[...],
                      preferred_element_type=jnp.float32)
        znew = z + jnp.tanh(agg) * _DT

        @pl.when(s < _STEPS)
        def _():
            out_ref[pl.ds(row, _BN), :] = znew
            part = jnp.dot(inct_ref[i], znew.astype(jnp.bfloat16),
                           preferred_element_type=jnp.float32)

            @pl.when(i == 0)
            def _():
                eacc_ref[...] = part

            @pl.when(i > 0)
            def _():
                eacc_ref[...] = eacc_ref[...] + part

            @pl.when(i == _NB - 1)
            def _():
                e2 = jnp.dot((eacc_ref[...] + c_ref[...]) * inv_e_ref[...],
                             w_ref[...],
                             preferred_element_type=jnp.float32) + b_ref[...]
                e2_ref[...] = e2.astype(jnp.bfloat16)
                c_ref[...] = c_ref[...] + q_ref[s]

        @pl.when(s == _STEPS)
        def _():
            out_ref[pl.ds(row, _BN), :] = (
                znew + csum_ref[pl.ds(row, _BN), :])


def kernel(node_features, incidence, W, b, log_sigma, dW):
    out = pl.pallas_call(
        _sde_kernel,
        grid=(_STEPS + 1, _NB),
        in_specs=[
            pl.BlockSpec((_D, _D), lambda s, i: (0, 0)),
            pl.BlockSpec((1, _D), lambda s, i: (0, 0)),
            pl.BlockSpec((1, _D), lambda s, i: (0, 0)),
            pl.BlockSpec((_BN, _M),
                         lambda s, i: (jnp.where(s == 0, i, _NB - 1), 0)),
            pl.BlockSpec((_BN, _D),
                         lambda s, i: (jnp.where(s == 0, i, _NB - 1), 0)),
            pl.BlockSpec(memory_space=pl.ANY),
        ],
        out_specs=pl.BlockSpec((_N, _D), lambda s, i: (0, 0)),
        out_shape=jax.ShapeDtypeStruct((_N, _D), jnp.float32),
        scratch_shapes=[
            pltpu.VMEM((_NB, _BN, _M), jnp.bfloat16),
            pltpu.VMEM((_NB, _M, _BN), jnp.bfloat16),
            pltpu.VMEM((_M, _D), jnp.bfloat16),
            pltpu.VMEM((_M, _D), jnp.float32),
            pltpu.VMEM((_STEPS, _M, _D), jnp.float32),
            pltpu.VMEM((_M, _D), jnp.float32),
            pltpu.VMEM((_N, _D), jnp.float32),
            pltpu.VMEM((_M, 1), jnp.float32),
            pltpu.VMEM((2, _STEPS, _BN, _D), jnp.float32),
            pltpu.SemaphoreType.DMA((2,)),
        ],
        compiler_params=pltpu.CompilerParams(
            dimension_semantics=("arbitrary", "arbitrary"),
            vmem_limit_bytes=100 * 1024 * 1024,
        ),
    )(W, b.reshape(1, _D), log_sigma.reshape(1, _D), incidence,
      node_features,
      dW.reshape(_STEPS, _N, _D).transpose(1, 0, 2).reshape(
          _NB, _BN, _STEPS, _D))
    return out.reshape(1, _N * _D)


# speedup vs baseline: 2.2322x; 1.7483x over previous
import jax
import jax.numpy as jnp
from jax.experimental import pallas as pl
from jax.experimental.pallas import tpu as pltpu

_N = 10000
_M = 512
_D = 128
_STEPS = 5
_DT = 0.2
_BN = 2000
_NB = _N // _BN
_T = _STEPS * _NB


def _noise_copy(dw_hbm, stage_ref, sem, t):
    blk = jax.lax.rem(t, _NB)
    step = jax.lax.div(t, _NB)
    buf = jax.lax.rem(t, 2)
    return pltpu.make_async_copy(
        dw_hbm.at[blk, :, step, :], stage_ref.at[buf], sem.at[buf])


def _sde_kernel(w_ref, b_ref, ls_ref, inc_ref, x0_ref, dw_hbm, out_ref,
                incb_ref, inct_ref, e2_ref, eacc_ref, inv_e_ref,
                stage_ref, sem):
    s = pl.program_id(0)
    i = pl.program_id(1)
    row = i * _BN

    @pl.when(s == 0)
    def _prologue():
        @pl.when(i == 0)
        def _():
            _noise_copy(dw_hbm, stage_ref, sem, 0).start()
            _noise_copy(dw_hbm, stage_ref, sem, 1).start()

        inc_f = inc_ref[...]
        x0 = x0_ref[...]
        out_ref[pl.ds(row, _BN), :] = x0
        inv_v = 1.0 / (jnp.sum(inc_f, axis=1)[:, None] + 1e-6)
        col = jnp.sum(inc_f, axis=0)[:, None]

        @pl.when(i == 0)
        def _():
            inv_e_ref[...] = col

        @pl.when(i > 0)
        def _():
            inv_e_ref[...] = inv_e_ref[...] + col

        @pl.when(i == _NB - 1)
        def _():
            inv_e_ref[...] = 1.0 / (inv_e_ref[...] + 1e-6)

        incb_ref[i] = (inc_f * inv_v).astype(jnp.bfloat16)
        inct = jnp.transpose(inc_f.astype(jnp.bfloat16))
        inct_ref[i] = inct

        part = jnp.dot(inct, x0.astype(jnp.bfloat16),
                       preferred_element_type=jnp.float32)

        @pl.when(i == 0)
        def _():
            eacc_ref[...] = part

        @pl.when(i > 0)
        def _():
            eacc_ref[...] = eacc_ref[...] + part

        @pl.when(i == _NB - 1)
        def _():
            e2 = jnp.dot(eacc_ref[...] * inv_e_ref[...], w_ref[...],
                         preferred_element_type=jnp.float32) + b_ref[...]
            e2_ref[...] = e2.astype(jnp.bfloat16)

    @pl.when(s > 0)
    def _update():
        t = (s - 1) * _NB + i
        _noise_copy(dw_hbm, stage_ref, sem, t).wait()
        noise = stage_ref[jax.lax.rem(t, 2)]
        sigma = jnp.exp(ls_ref[...])
        y = out_ref[pl.ds(row, _BN), :]
        agg = jnp.dot(incb_ref[i], e2_ref[...],
                      preferred_element_type=jnp.float32)
        ynew = y + jnp.tanh(agg) * _DT + noise * sigma
        out_ref[pl.ds(row, _BN), :] = ynew

        @pl.when(t + 2 < _T)
        def _():
            _noise_copy(dw_hbm, stage_ref, sem, t + 2).start()

        @pl.when(s < _STEPS)
        def _():
            part = jnp.dot(inct_ref[i], ynew.astype(jnp.bfloat16),
                           preferred_element_type=jnp.float32)

            @pl.when(i == 0)
            def _():
                eacc_ref[...] = part

            @pl.when(i > 0)
            def _():
                eacc_ref[...] = eacc_ref[...] + part

            @pl.when(i == _NB - 1)
            def _():
                e2 = jnp.dot(eacc_ref[...] * inv_e_ref[...], w_ref[...],
                             preferred_element_type=jnp.float32) + b_ref[...]
                e2_ref[...] = e2.astype(jnp.bfloat16)


def kernel(node_features, incidence, W, b, log_sigma, dW):
    out = pl.pallas_call(
        _sde_kernel,
        grid=(_STEPS + 1, _NB),
        in_specs=[
            pl.BlockSpec((_D, _D), lambda s, i: (0, 0)),
            pl.BlockSpec((1, _D), lambda s, i: (0, 0)),
            pl.BlockSpec((1, _D), lambda s, i: (0, 0)),
            pl.BlockSpec((_BN, _M),
                         lambda s, i: (jnp.where(s == 0, i, _NB - 1), 0)),
            pl.BlockSpec((_BN, _D),
                         lambda s, i: (jnp.where(s == 0, i, _NB - 1), 0)),
            pl.BlockSpec(memory_space=pl.ANY),
        ],
        out_specs=pl.BlockSpec((_N, _D), lambda s, i: (0, 0)),
        out_shape=jax.ShapeDtypeStruct((_N, _D), jnp.float32),
        scratch_shapes=[
            pltpu.VMEM((_NB, _BN, _M), jnp.bfloat16),
            pltpu.VMEM((_NB, _M, _BN), jnp.bfloat16),
            pltpu.VMEM((_M, _D), jnp.bfloat16),
            pltpu.VMEM((_M, _D), jnp.float32),
            pltpu.VMEM((_M, 1), jnp.float32),
            pltpu.VMEM((2, _BN, _D), jnp.float32),
            pltpu.SemaphoreType.DMA((2,)),
        ],
        compiler_params=pltpu.CompilerParams(
            dimension_semantics=("arbitrary", "arbitrary"),
            vmem_limit_bytes=100 * 1024 * 1024,
        ),
    )(W, b.reshape(1, _D), log_sigma.reshape(1, _D), incidence,
      node_features,
      dW.reshape(_STEPS, _N, _D).transpose(1, 0, 2).reshape(
          _NB, _BN, _STEPS, _D))
    return out.reshape(1, _N * _D)


# speedup vs baseline: 2.2507x; 1.0083x over previous
import jax
import jax.numpy as jnp
from jax.experimental import pallas as pl
from jax.experimental.pallas import tpu as pltpu

_N = 10000
_M = 512
_D = 128
_STEPS = 5
_DT = 0.2
_BN = 2000
_NB = _N // _BN
_T = _STEPS * _NB


def _noise_copy(dw_hbm, stage_ref, sem, t):
    blk = jax.lax.rem(t, _NB)
    step = jax.lax.div(t, _NB)
    buf = jax.lax.rem(t, 4)
    return pltpu.make_async_copy(
        dw_hbm.at[blk, :, step, :], stage_ref.at[buf], sem.at[buf])


def _sde_kernel(w_ref, b_ref, ls_ref, inc_ref, x0_ref, dw_hbm, out_ref,
                incb_ref, inct_ref, e2_ref, eacc_ref, inv_e_ref,
                stage_ref, sem):
    s = pl.program_id(0)
    i = pl.program_id(1)
    row = i * _BN

    @pl.when(s == 0)
    def _prologue():
        @pl.when(i == 0)
        def _():
            _noise_copy(dw_hbm, stage_ref, sem, 0).start()
            _noise_copy(dw_hbm, stage_ref, sem, 1).start()
            _noise_copy(dw_hbm, stage_ref, sem, 2).start()
            _noise_copy(dw_hbm, stage_ref, sem, 3).start()

        inc_f = inc_ref[...]
        x0 = x0_ref[...]
        out_ref[pl.ds(row, _BN), :] = x0
        inv_v = 1.0 / (jnp.sum(inc_f, axis=1)[:, None] + 1e-6)
        col = jnp.sum(inc_f, axis=0)[:, None]

        @pl.when(i == 0)
        def _():
            inv_e_ref[...] = col

        @pl.when(i > 0)
        def _():
            inv_e_ref[...] = inv_e_ref[...] + col

        @pl.when(i == _NB - 1)
        def _():
            inv_e_ref[...] = 1.0 / (inv_e_ref[...] + 1e-6)

        incb_ref[i] = (inc_f * inv_v).astype(jnp.bfloat16)
        inct = jnp.transpose(inc_f.astype(jnp.bfloat16))
        inct_ref[i] = inct

        part = jnp.dot(inct, x0.astype(jnp.bfloat16),
                       preferred_element_type=jnp.float32)

        @pl.when(i == 0)
        def _():
            eacc_ref[...] = part

        @pl.when(i > 0)
        def _():
            eacc_ref[...] = eacc_ref[...] + part

        @pl.when(i == _NB - 1)
        def _():
            e2 = jnp.dot(eacc_ref[...] * inv_e_ref[...], w_ref[...],
                         preferred_element_type=jnp.float32) + b_ref[...]
            e2_ref[...] = e2.astype(jnp.bfloat16)

    @pl.when(s > 0)
    def _update():
        t = (s - 1) * _NB + i
        _noise_copy(dw_hbm, stage_ref, sem, t).wait()
        noise = stage_ref[jax.lax.rem(t, 4)]
        sigma = jnp.exp(ls_ref[...])
        y = out_ref[pl.ds(row, _BN), :]
        agg = jnp.dot(incb_ref[i], e2_ref[...],
                      preferred_element_type=jnp.float32)
        ynew = y + jnp.tanh(agg) * _DT + noise * sigma
        out_ref[pl.ds(row, _BN), :] = ynew

        @pl.when(t + 4 < _T)
        def _():
            _noise_copy(dw_hbm, stage_ref, sem, t + 4).start()

        @pl.when(s < _STEPS)
        def _():
            part = jnp.dot(inct_ref[i], ynew.astype(jnp.bfloat16),
                           preferred_element_type=jnp.float32)

            @pl.when(i == 0)
            def _():
                eacc_ref[...] = part

            @pl.when(i > 0)
            def _():
                eacc_ref[...] = eacc_ref[...] + part

            @pl.when(i == _NB - 1)
            def _():
                e2 = jnp.dot(eacc_ref[...] * inv_e_ref[...], w_ref[...],
                             preferred_element_type=jnp.float32) + b_ref[...]
                e2_ref[...] = e2.astype(jnp.bfloat16)


def kernel(node_features, incidence, W, b, log_sigma, dW):
    out = pl.pallas_call(
        _sde_kernel,
        grid=(_STEPS + 1, _NB),
        in_specs=[
            pl.BlockSpec((_D, _D), lambda s, i: (0, 0)),
            pl.BlockSpec((1, _D), lambda s, i: (0, 0)),
            pl.BlockSpec((1, _D), lambda s, i: (0, 0)),
            pl.BlockSpec((_BN, _M),
                         lambda s, i: (jnp.where(s == 0, i, _NB - 1), 0)),
            pl.BlockSpec((_BN, _D),
                         lambda s, i: (jnp.where(s == 0, i, _NB - 1), 0)),
            pl.BlockSpec(memory_space=pl.ANY),
        ],
        out_specs=pl.BlockSpec((_N, _D), lambda s, i: (0, 0)),
        out_shape=jax.ShapeDtypeStruct((_N, _D), jnp.float32),
        scratch_shapes=[
            pltpu.VMEM((_NB, _BN, _M), jnp.bfloat16),
            pltpu.VMEM((_NB, _M, _BN), jnp.bfloat16),
            pltpu.VMEM((_M, _D), jnp.bfloat16),
            pltpu.VMEM((_M, _D), jnp.float32),
            pltpu.VMEM((_M, 1), jnp.float32),
            pltpu.VMEM((4, _BN, _D), jnp.float32),
            pltpu.SemaphoreType.DMA((4,)),
        ],
        compiler_params=pltpu.CompilerParams(
            dimension_semantics=("arbitrary", "arbitrary"),
            vmem_limit_bytes=100 * 1024 * 1024,
        ),
    )(W, b.reshape(1, _D), log_sigma.reshape(1, _D), incidence,
      node_features,
      dW.reshape(_STEPS, _N, _D).transpose(1, 0, 2).reshape(
          _NB, _BN, _STEPS, _D))
    return out.reshape(1, _N * _D)
